# baseline (speedup 1.0000x reference)
import functools

import jax
import jax.numpy as jnp
from jax.experimental import pallas as pl
from jax.experimental.pallas import tpu as pltpu

_PAD = 16
_COLS = 64
_B = 6
_RB = 16


def _fused_kernel(x_ref, br_ref, wup_ref, bup_ref, wuc_ref, buc_ref,
                  g_ref, bln_ref, w1_ref, b1_ref, w2_ref, b2_ref,
                  o_ref, up_sc, cat0, cat1, cat2, tp0, tp1, tp2,
                  *, slope, eps):
    f32 = jnp.float32
    bf16 = jnp.bfloat16
    B, H, W, Cin = x_ref.shape
    _, Ho, Wo, C = o_ref.shape
    Ctot = cat0.shape[-1]
    M = _RB * Wo
    NCH = Ho // _RB
    cats = (cat0, cat1, cat2)
    tps = (tp0, tp1, tp2)

    wup = [jnp.concatenate(
        [wup_ref[(2 * di) * Cin:(2 * di + 1) * Cin, :],
         wup_ref[(2 * di + 1) * Cin:(2 * di + 2) * Cin, :]],
        axis=1).astype(bf16) for di in range(2)]
    bup2 = jnp.concatenate([bup_ref[...], bup_ref[...]], axis=1)
    wuc = [wuc_ref[k * Ctot:(k + 1) * Ctot, :].astype(bf16) for k in range(9)]

    def pair_taps(w_ref):
        ws = [jnp.concatenate(
            [w_ref[2 * j * C:(2 * j + 1) * C, :],
             w_ref[(2 * j + 1) * C:(2 * j + 2) * C, :]],
            axis=0).astype(bf16) for j in range(4)]
        ws.append(w_ref[8 * C:9 * C, :].astype(bf16))
        return ws

    w1 = [w1_ref[k * C:(k + 1) * C, :].astype(bf16) for k in range(9)]
    w2 = [w2_ref[k * C:(k + 1) * C, :].astype(bf16) for k in range(9)]

    for group, Cc in ((cats, Ctot), (tps, C)):
        zrow = jnp.zeros((B, 1, Wo, Cc), bf16)
        zcol = jnp.zeros((B, Ho + 2, 1, Cc), bf16)
        for sc in group:
            sc[:, 0:1, _PAD:_PAD + Wo, :] = zrow
            sc[:, Ho + 1:Ho + 2, _PAD:_PAD + Wo, :] = zrow
        group[0][:, :, _PAD:_PAD + 1, :] = zcol
        group[2][:, :, _PAD + Wo - 1:_PAD + Wo, :] = zcol

    for b in range(B):
        x2d = x_ref[b].reshape(H * W, Cin).astype(bf16)
        for di in range(2):
            yk = (jnp.dot(x2d, wup[di], preferred_element_type=f32)
                  + bup2)
            up_sc[b, :, di] = yk.reshape(H, W, 2 * C).astype(bf16)
        upv = up_sc[b].reshape(Ho, Wo, C)
        brv = br_ref[b].astype(bf16)
        for kx in range(3):
            p = _PAD + 1 - kx
            cats[kx][b, 1:Ho + 1, p:p + Wo, 0:C] = upv
            cats[kx][b, 1:Ho + 1, p:p + Wo, C:Ctot] = brv

    def window(b, r, srcs, Cc, k):
        ky, kx = divmod(k, 3)
        row = r * _RB + ky
        return srcs[kx][b, row:row + _RB, _PAD:_PAD + Wo, :].reshape(M, Cc)

    def conv_chunk(b, r, srcs, Cc, taps, b_ref):
        acc = jnp.broadcast_to(b_ref[...], (M, C)).astype(f32)
        for k in range(9):
            acc = acc + jnp.dot(window(b, r, srcs, Cc, k), taps[k],
                                preferred_element_type=f32)
        return acc

    def conv_chunk_paired(b, r, srcs, taps, b_ref):
        acc = jnp.broadcast_to(b_ref[...], (M, C)).astype(f32)
        for j in range(4):
            a = jnp.concatenate(
                [window(b, r, srcs, C, 2 * j),
                 window(b, r, srcs, C, 2 * j + 1)], axis=1)
            acc = acc + jnp.dot(a, taps[j], preferred_element_type=f32)
        return acc + jnp.dot(window(b, r, srcs, C, 8), taps[4],
                             preferred_element_type=f32)

    for b in range(B):
        for r in range(NCH):
            y = conv_chunk(b, r, cats, Ctot, wuc, buc_ref)
            o_ref[b, r * _RB:(r + 1) * _RB] = y.reshape(_RB, Wo, C)
            mu = jnp.mean(y, axis=-1, keepdims=True)
            var = jnp.mean((y - mu) ** 2, axis=-1, keepdims=True)
            t = ((y - mu) * jax.lax.rsqrt(var + eps) * g_ref[...]
                 + bln_ref[...])
            tv = t.reshape(_RB, Wo, C).astype(bf16)
            for kx in range(3):
                p = _PAD + 1 - kx
                tps[kx][b, 1 + r * _RB:1 + (r + 1) * _RB, p:p + Wo, :] = tv

    hvs = []
    for b in range(B):
        for r in range(NCH):
            h = conv_chunk(b, r, tps, C, w1, b1_ref)
            h = jnp.where(h >= 0, h, h * slope)
            hvs.append(h.reshape(_RB, Wo, C).astype(bf16))
    for b in range(B):
        for r in range(NCH):
            hv = hvs[b * NCH + r]
            for kx in range(3):
                p = _PAD + 1 - kx
                tps[kx][b, 1 + r * _RB:1 + (r + 1) * _RB, p:p + Wo, :] = hv

    for b in range(B):
        for r in range(NCH):
            h = conv_chunk(b, r, tps, C, w2, b2_ref)
            sl = slice(r * _RB, (r + 1) * _RB)
            o_ref[b, sl] = (o_ref[b, sl]
                            + h.reshape(_RB, Wo, C)).astype(o_ref.dtype)


def kernel(x, bridge, w_up, b_up, w_uc, b_uc, ln_g, ln_b, w1, b1, w2, b2):
    N, H, W, Cin = x.shape
    C = w_up.shape[-1]
    Cb = bridge.shape[-1]
    Ho, Wo = 2 * H, 2 * W
    Ctot = C + Cb

    wup_p = w_up.reshape(4 * Cin, C)
    bup_p = b_up.reshape(1, C)
    wuc_p = w_uc.reshape(9 * Ctot, C)
    buc_p = b_uc.reshape(1, C)
    g_p = ln_g.reshape(1, C)
    bln_p = ln_b.reshape(1, C)
    w1_p = w1.reshape(9 * C, C)
    b1_p = b1.reshape(1, C)
    w2_p = w2.reshape(9 * C, C)
    b2_p = b2.reshape(1, C)

    B = _B if N % _B == 0 else 1
    img = lambda n: (n, 0, 0, 0)
    wgt = lambda n: (0, 0)
    return pl.pallas_call(
        functools.partial(_fused_kernel, slope=0.2, eps=1e-5),
        out_shape=jax.ShapeDtypeStruct((N, Ho, Wo, C), x.dtype),
        grid=(N // B,),
        in_specs=[
            pl.BlockSpec((B, H, W, Cin), img),
            pl.BlockSpec((B, Ho, Wo, Cb), img),
            pl.BlockSpec((4 * Cin, C), wgt),
            pl.BlockSpec((1, C), wgt),
            pl.BlockSpec((9 * Ctot, C), wgt),
            pl.BlockSpec((1, C), wgt),
            pl.BlockSpec((1, C), wgt),
            pl.BlockSpec((1, C), wgt),
            pl.BlockSpec((9 * C, C), wgt),
            pl.BlockSpec((1, C), wgt),
            pl.BlockSpec((9 * C, C), wgt),
            pl.BlockSpec((1, C), wgt),
        ],
        out_specs=pl.BlockSpec((B, Ho, Wo, C), img),
        scratch_shapes=[
            pltpu.VMEM((B, H, 2, W, 2 * C), jnp.bfloat16),
            pltpu.VMEM((B, Ho + 2, _COLS, Ctot), jnp.bfloat16),
            pltpu.VMEM((B, Ho + 2, _COLS, Ctot), jnp.bfloat16),
            pltpu.VMEM((B, Ho + 2, _COLS, Ctot), jnp.bfloat16),
            pltpu.VMEM((B, Ho + 2, _COLS, C), jnp.bfloat16),
            pltpu.VMEM((B, Ho + 2, _COLS, C), jnp.bfloat16),
            pltpu.VMEM((B, Ho + 2, _COLS, C), jnp.bfloat16),
        ],
        compiler_params=pltpu.CompilerParams(
            dimension_semantics=("arbitrary",)),
    )(x, bridge, wup_p, bup_p, wuc_p, buc_p, g_p, bln_p, w1_p, b1_p, w2_p, b2_p)


# speedup vs baseline: 1.0105x; 1.0105x over previous
import functools

import jax
import jax.numpy as jnp
from jax.experimental import pallas as pl
from jax.experimental.pallas import tpu as pltpu

_PAD = 16
_COLS = 64
_B = 6
_RB = 8


def _fused_kernel(x_ref, br_ref, wup_ref, bup_ref, wuc_ref, buc_ref,
                  g_ref, bln_ref, w1_ref, b1_ref, w2_ref, b2_ref,
                  o_ref, up_sc, cat0, cat1, cat2, tpP, tp2,
                  *, slope, eps):
    f32 = jnp.float32
    bf16 = jnp.bfloat16
    B, H, W, Cin = x_ref.shape
    _, Ho, Wo, C = o_ref.shape
    Ctot = cat0.shape[-1]
    M = _RB * Wo
    NCH = Ho // _RB
    cats = (cat0, cat1, cat2)

    wup = [jnp.concatenate(
        [wup_ref[(2 * di) * Cin:(2 * di + 1) * Cin, :],
         wup_ref[(2 * di + 1) * Cin:(2 * di + 2) * Cin, :]],
        axis=1).astype(bf16) for di in range(2)]
    bup2 = jnp.concatenate([bup_ref[...], bup_ref[...]], axis=1)
    wuc = [wuc_ref[k * Ctot:(k + 1) * Ctot, :].astype(bf16) for k in range(9)]

    def pair_taps(w_ref):
        pairs = [jnp.concatenate(
            [w_ref[(3 * ky) * C:(3 * ky + 1) * C, :],
             w_ref[(3 * ky + 1) * C:(3 * ky + 2) * C, :]],
            axis=0).astype(bf16) for ky in range(3)]
        singles = [w_ref[(3 * ky + 2) * C:(3 * ky + 3) * C, :].astype(bf16)
                   for ky in range(3)]
        return pairs, singles

    w1p, w1s = pair_taps(w1_ref)
    w2p, w2s = pair_taps(w2_ref)

    zrow = jnp.zeros((B, 1, Wo, Ctot), bf16)
    zcol = jnp.zeros((B, Ho + 2, 1, Ctot), bf16)
    for sc in cats:
        sc[:, 0:1, _PAD:_PAD + Wo, :] = zrow
        sc[:, Ho + 1:Ho + 2, _PAD:_PAD + Wo, :] = zrow
    cats[0][:, :, _PAD:_PAD + 1, :] = zcol
    cats[2][:, :, _PAD + Wo - 1:_PAD + Wo, :] = zcol
    tpP[:, 0:1, _PAD:_PAD + Wo, :] = jnp.zeros((B, 1, Wo, 2 * C), bf16)
    tpP[:, Ho + 1:Ho + 2, _PAD:_PAD + Wo, :] = jnp.zeros((B, 1, Wo, 2 * C),
                                                         bf16)
    tpP[:, :, _PAD:_PAD + 1, 0:C] = jnp.zeros((B, Ho + 2, 1, C), bf16)
    tp2[:, 0:1, _PAD:_PAD + Wo, :] = jnp.zeros((B, 1, Wo, C), bf16)
    tp2[:, Ho + 1:Ho + 2, _PAD:_PAD + Wo, :] = jnp.zeros((B, 1, Wo, C), bf16)
    tp2[:, :, _PAD + Wo - 1:_PAD + Wo, :] = jnp.zeros((B, Ho + 2, 1, C),
                                                      bf16)

    for b in range(B):
        x2d = x_ref[b].reshape(H * W, Cin).astype(bf16)
        for di in range(2):
            yk = (jnp.dot(x2d, wup[di], preferred_element_type=f32)
                  + bup2)
            up_sc[b, :, di] = yk.reshape(H, W, 2 * C).astype(bf16)
        upv = up_sc[b].reshape(Ho, Wo, C)
        brv = br_ref[b].astype(bf16)
        for kx in range(3):
            p = _PAD + 1 - kx
            cats[kx][b, 1:Ho + 1, p:p + Wo, 0:C] = upv
            cats[kx][b, 1:Ho + 1, p:p + Wo, C:Ctot] = brv

    def conv_chunk_uc(b, r, b_ref):
        acc = jnp.broadcast_to(b_ref[...], (M, C)).astype(f32)
        for ky in range(3):
            row = r * _RB + ky
            for kx in range(3):
                a = (cats[kx][b, row:row + _RB, _PAD:_PAD + Wo, :]
                     .reshape(M, Ctot))
                acc = acc + jnp.dot(a, wuc[ky * 3 + kx],
                                    preferred_element_type=f32)
        return acc

    def conv_chunk_pp(b, r, pairs, singles, b_ref):
        acc = jnp.broadcast_to(b_ref[...], (M, C)).astype(f32)
        for ky in range(3):
            row = r * _RB + ky
            aP = (tpP[b, row:row + _RB, _PAD:_PAD + Wo, :]
                  .reshape(M, 2 * C))
            acc = acc + jnp.dot(aP, pairs[ky], preferred_element_type=f32)
            a2 = (tp2[b, row:row + _RB, _PAD:_PAD + Wo, :]
                  .reshape(M, C))
            acc = acc + jnp.dot(a2, singles[ky], preferred_element_type=f32)
        return acc

    def store_th(b, r, tv):
        rows = slice(1 + r * _RB, 1 + (r + 1) * _RB)
        tpP[b, rows, _PAD + 1:_PAD + 1 + Wo, 0:C] = tv
        tpP[b, rows, _PAD:_PAD + Wo, C:2 * C] = tv
        tp2[b, rows, _PAD - 1:_PAD - 1 + Wo, :] = tv

    for b in range(B):
        for r in range(NCH):
            y = conv_chunk_uc(b, r, buc_ref)
            o_ref[b, r * _RB:(r + 1) * _RB] = y.reshape(_RB, Wo, C)
            mu = jnp.mean(y, axis=-1, keepdims=True)
            var = jnp.mean((y - mu) ** 2, axis=-1, keepdims=True)
            t = ((y - mu) * jax.lax.rsqrt(var + eps) * g_ref[...]
                 + bln_ref[...])
            store_th(b, r, t.reshape(_RB, Wo, C).astype(bf16))

    hvs = []
    for b in range(B):
        for r in range(NCH):
            h = conv_chunk_pp(b, r, w1p, w1s, b1_ref)
            h = jnp.where(h >= 0, h, h * slope)
            hvs.append(h.reshape(_RB, Wo, C).astype(bf16))
    for b in range(B):
        for r in range(NCH):
            store_th(b, r, hvs[b * NCH + r])

    for b in range(B):
        for r in range(NCH):
            h = conv_chunk_pp(b, r, w2p, w2s, b2_ref)
            sl = slice(r * _RB, (r + 1) * _RB)
            o_ref[b, sl] = (o_ref[b, sl]
                            + h.reshape(_RB, Wo, C)).astype(o_ref.dtype)


def kernel(x, bridge, w_up, b_up, w_uc, b_uc, ln_g, ln_b, w1, b1, w2, b2):
    N, H, W, Cin = x.shape
    C = w_up.shape[-1]
    Cb = bridge.shape[-1]
    Ho, Wo = 2 * H, 2 * W
    Ctot = C + Cb

    wup_p = w_up.reshape(4 * Cin, C)
    bup_p = b_up.reshape(1, C)
    wuc_p = w_uc.reshape(9 * Ctot, C)
    buc_p = b_uc.reshape(1, C)
    g_p = ln_g.reshape(1, C)
    bln_p = ln_b.reshape(1, C)
    w1_p = w1.reshape(9 * C, C)
    b1_p = b1.reshape(1, C)
    w2_p = w2.reshape(9 * C, C)
    b2_p = b2.reshape(1, C)

    B = _B if N % _B == 0 else 1
    img = lambda n: (n, 0, 0, 0)
    wgt = lambda n: (0, 0)
    return pl.pallas_call(
        functools.partial(_fused_kernel, slope=0.2, eps=1e-5),
        out_shape=jax.ShapeDtypeStruct((N, Ho, Wo, C), x.dtype),
        grid=(N // B,),
        in_specs=[
            pl.BlockSpec((B, H, W, Cin), img),
            pl.BlockSpec((B, Ho, Wo, Cb), img),
            pl.BlockSpec((4 * Cin, C), wgt),
            pl.BlockSpec((1, C), wgt),
            pl.BlockSpec((9 * Ctot, C), wgt),
            pl.BlockSpec((1, C), wgt),
            pl.BlockSpec((1, C), wgt),
            pl.BlockSpec((1, C), wgt),
            pl.BlockSpec((9 * C, C), wgt),
            pl.BlockSpec((1, C), wgt),
            pl.BlockSpec((9 * C, C), wgt),
            pl.BlockSpec((1, C), wgt),
        ],
        out_specs=pl.BlockSpec((B, Ho, Wo, C), img),
        scratch_shapes=[
            pltpu.VMEM((B, H, 2, W, 2 * C), jnp.bfloat16),
            pltpu.VMEM((B, Ho + 2, _COLS, Ctot), jnp.bfloat16),
            pltpu.VMEM((B, Ho + 2, _COLS, Ctot), jnp.bfloat16),
            pltpu.VMEM((B, Ho + 2, _COLS, Ctot), jnp.bfloat16),
            pltpu.VMEM((B, Ho + 2, _COLS, 2 * C), jnp.bfloat16),
            pltpu.VMEM((B, Ho + 2, _COLS, C), jnp.bfloat16),
        ],
        compiler_params=pltpu.CompilerParams(
            dimension_semantics=("arbitrary",)),
    )(x, bridge, wup_p, bup_p, wuc_p, buc_p, g_p, bln_p, w1_p, b1_p, w2_p, b2_p)


# speedup vs baseline: 1.0344x; 1.0237x over previous
import functools

import jax
import jax.numpy as jnp
from jax.experimental import pallas as pl
from jax.experimental.pallas import tpu as pltpu

_PAD = 16
_COLS = 64
_B = 6
_RB = 8


def _fused_kernel(x_ref, br_ref, wup_ref, bup_ref, wuc_ref, buc_ref,
                  g_ref, bln_ref, w1_ref, b1_ref, w2_ref, b2_ref,
                  o_ref, up_sc, cat0, cat1, cat2, tp0, tp1, tp2,
                  *, slope, eps):
    f32 = jnp.float32
    bf16 = jnp.bfloat16
    B, H, W, Cin = x_ref.shape
    _, Ho, Wo, C = o_ref.shape
    Ctot = cat0.shape[-1]
    M = _RB * Wo
    NCH = Ho // _RB
    cats = (cat0, cat1, cat2)
    tps = (tp0, tp1, tp2)

    wup = [jnp.concatenate(
        [wup_ref[(2 * di) * Cin:(2 * di + 1) * Cin, :],
         wup_ref[(2 * di + 1) * Cin:(2 * di + 2) * Cin, :]],
        axis=1).astype(bf16) for di in range(2)]
    bup2 = jnp.concatenate([bup_ref[...], bup_ref[...]], axis=1)
    wuc = [wuc_ref[k * Ctot:(k + 1) * Ctot, :].astype(bf16) for k in range(9)]
    w1 = [w1_ref[k * C:(k + 1) * C, :].astype(bf16) for k in range(9)]
    w2 = [w2_ref[k * C:(k + 1) * C, :].astype(bf16) for k in range(9)]

    for group, Cc in ((cats, Ctot), (tps, C)):
        zrow = jnp.zeros((B, 1, Wo, Cc), bf16)
        zcol = jnp.zeros((B, Ho + 2, 1, Cc), bf16)
        for sc in group:
            sc[:, 0:1, _PAD:_PAD + Wo, :] = zrow
            sc[:, Ho + 1:Ho + 2, _PAD:_PAD + Wo, :] = zrow
        group[0][:, :, _PAD:_PAD + 1, :] = zcol
        group[2][:, :, _PAD + Wo - 1:_PAD + Wo, :] = zcol

    for b in range(B):
        x2d = x_ref[b].reshape(H * W, Cin).astype(bf16)
        for di in range(2):
            yk = (jnp.dot(x2d, wup[di], preferred_element_type=f32)
                  + bup2)
            up_sc[b, :, di] = yk.reshape(H, W, 2 * C).astype(bf16)
        upv = up_sc[b].reshape(Ho, Wo, C)
        brv = br_ref[b].astype(bf16)
        for kx in range(3):
            p = _PAD + 1 - kx
            cats[kx][b, 1:Ho + 1, p:p + Wo, 0:C] = upv
            cats[kx][b, 1:Ho + 1, p:p + Wo, C:Ctot] = brv

    def conv_chunk(b, r, srcs, Cc, taps, b_ref):
        acc = jnp.broadcast_to(b_ref[...], (M, C)).astype(f32)
        for ky in range(3):
            row = r * _RB + ky
            for kx in range(3):
                a = (srcs[kx][b, row:row + _RB, _PAD:_PAD + Wo, :]
                     .reshape(M, Cc))
                acc = acc + jnp.dot(a, taps[ky * 3 + kx],
                                    preferred_element_type=f32)
        return acc

    for b in range(B):
        for r in range(NCH):
            y = conv_chunk(b, r, cats, Ctot, wuc, buc_ref)
            o_ref[b, r * _RB:(r + 1) * _RB] = y.reshape(_RB, Wo, C)
            mu = jnp.mean(y, axis=-1, keepdims=True)
            var = jnp.mean((y - mu) ** 2, axis=-1, keepdims=True)
            t = ((y - mu) * jax.lax.rsqrt(var + eps) * g_ref[...]
                 + bln_ref[...])
            tv = t.reshape(_RB, Wo, C).astype(bf16)
            for kx in range(3):
                p = _PAD + 1 - kx
                tps[kx][b, 1 + r * _RB:1 + (r + 1) * _RB, p:p + Wo, :] = tv

    hvs = []
    for b in range(B):
        for r in range(NCH):
            h = conv_chunk(b, r, tps, C, w1, b1_ref)
            h = jnp.where(h >= 0, h, h * slope)
            hvs.append(h.reshape(_RB, Wo, C).astype(bf16))
    for b in range(B):
        for r in range(NCH):
            hv = hvs[b * NCH + r]
            for kx in range(3):
                p = _PAD + 1 - kx
                tps[kx][b, 1 + r * _RB:1 + (r + 1) * _RB, p:p + Wo, :] = hv

    for b in range(B):
        for r in range(NCH):
            h = conv_chunk(b, r, tps, C, w2, b2_ref)
            sl = slice(r * _RB, (r + 1) * _RB)
            o_ref[b, sl] = (o_ref[b, sl]
                            + h.reshape(_RB, Wo, C)).astype(o_ref.dtype)


def kernel(x, bridge, w_up, b_up, w_uc, b_uc, ln_g, ln_b, w1, b1, w2, b2):
    N, H, W, Cin = x.shape
    C = w_up.shape[-1]
    Cb = bridge.shape[-1]
    Ho, Wo = 2 * H, 2 * W
    Ctot = C + Cb

    wup_p = w_up.reshape(4 * Cin, C)
    bup_p = b_up.reshape(1, C)
    wuc_p = w_uc.reshape(9 * Ctot, C)
    buc_p = b_uc.reshape(1, C)
    g_p = ln_g.reshape(1, C)
    bln_p = ln_b.reshape(1, C)
    w1_p = w1.reshape(9 * C, C)
    b1_p = b1.reshape(1, C)
    w2_p = w2.reshape(9 * C, C)
    b2_p = b2.reshape(1, C)

    B = _B if N % _B == 0 else 1
    img = lambda n: (n, 0, 0, 0)
    wgt = lambda n: (0, 0)
    return pl.pallas_call(
        functools.partial(_fused_kernel, slope=0.2, eps=1e-5),
        out_shape=jax.ShapeDtypeStruct((N, Ho, Wo, C), x.dtype),
        grid=(N // B,),
        in_specs=[
            pl.BlockSpec((B, H, W, Cin), img),
            pl.BlockSpec((B, Ho, Wo, Cb), img),
            pl.BlockSpec((4 * Cin, C), wgt),
            pl.BlockSpec((1, C), wgt),
            pl.BlockSpec((9 * Ctot, C), wgt),
            pl.BlockSpec((1, C), wgt),
            pl.BlockSpec((1, C), wgt),
            pl.BlockSpec((1, C), wgt),
            pl.BlockSpec((9 * C, C), wgt),
            pl.BlockSpec((1, C), wgt),
            pl.BlockSpec((9 * C, C), wgt),
            pl.BlockSpec((1, C), wgt),
        ],
        out_specs=pl.BlockSpec((B, Ho, Wo, C), img),
        scratch_shapes=[
            pltpu.VMEM((B, H, 2, W, 2 * C), jnp.bfloat16),
            pltpu.VMEM((B, Ho + 2, _COLS, Ctot), jnp.bfloat16),
            pltpu.VMEM((B, Ho + 2, _COLS, Ctot), jnp.bfloat16),
            pltpu.VMEM((B, Ho + 2, _COLS, Ctot), jnp.bfloat16),
            pltpu.VMEM((B, Ho + 2, _COLS, C), jnp.bfloat16),
            pltpu.VMEM((B, Ho + 2, _COLS, C), jnp.bfloat16),
            pltpu.VMEM((B, Ho + 2, _COLS, C), jnp.bfloat16),
        ],
        compiler_params=pltpu.CompilerParams(
            dimension_semantics=("arbitrary",)),
    )(x, bridge, wup_p, bup_p, wuc_p, buc_p, g_p, bln_p, w1_p, b1_p, w2_p, b2_p)


# speedup vs baseline: 1.0399x; 1.0053x over previous
import functools

import jax
import jax.numpy as jnp
from jax.experimental import pallas as pl
from jax.experimental.pallas import tpu as pltpu

_PAD = 16
_COLS = 64
_B = 6
_RB = 8


def _fused_kernel(x_ref, br_ref, wup_ref, bup_ref, wuc_ref, buc_ref,
                  g_ref, bln_ref, w1_ref, b1_ref, w2_ref, b2_ref,
                  o_ref, up_sc, cat0, cat1, cat2, tp0, tp1, tp2,
                  *, slope, eps):
    f32 = jnp.float32
    bf16 = jnp.bfloat16
    B, H, W, Cin = x_ref.shape
    _, Ho, Wo, C = o_ref.shape
    Ctot = cat0.shape[-1]
    M = _RB * Wo
    NCH = Ho // _RB
    cats = (cat0, cat1, cat2)
    tps = (tp0, tp1, tp2)

    wup = [jnp.concatenate(
        [wup_ref[(2 * di) * Cin:(2 * di + 1) * Cin, :],
         wup_ref[(2 * di + 1) * Cin:(2 * di + 2) * Cin, :]],
        axis=1).astype(bf16) for di in range(2)]
    bup2 = jnp.concatenate([bup_ref[...], bup_ref[...]], axis=1)
    wuc = [wuc_ref[k * Ctot:(k + 1) * Ctot, :].astype(bf16) for k in range(9)]
    w1 = [w1_ref[k * C:(k + 1) * C, :].astype(bf16) for k in range(9)]
    w2 = [w2_ref[k * C:(k + 1) * C, :].astype(bf16) for k in range(9)]

    for group, Cc in ((cats, Ctot), (tps, C)):
        zrow = jnp.zeros((B, 1, Wo, Cc), bf16)
        zcol = jnp.zeros((B, Ho + 2, 1, Cc), bf16)
        for sc in group:
            sc[:, 0:1, _PAD:_PAD + Wo, :] = zrow
            sc[:, Ho + 1:Ho + 2, _PAD:_PAD + Wo, :] = zrow
        group[0][:, :, _PAD:_PAD + 1, :] = zcol
        group[2][:, :, _PAD + Wo - 1:_PAD + Wo, :] = zcol

    for b in range(B):
        x2d = x_ref[b].reshape(H * W, Cin).astype(bf16)
        for di in range(2):
            yk = (jnp.dot(x2d, wup[di], preferred_element_type=f32)
                  + bup2)
            up_sc[b, :, di] = yk.reshape(H, W, 2 * C).astype(bf16)
        upv = up_sc[b].reshape(Ho, Wo, C)
        brv = br_ref[b].astype(bf16)
        for kx in range(3):
            p = _PAD + 1 - kx
            cats[kx][b, 1:Ho + 1, p:p + Wo, 0:C] = upv
            cats[kx][b, 1:Ho + 1, p:p + Wo, C:Ctot] = brv

    def conv_chunk(b, r, srcs, Cc, taps, b_ref):
        acc = jnp.broadcast_to(b_ref[...], (M, C)).astype(f32)
        for ky in range(3):
            row = r * _RB + ky
            for kx in range(3):
                a = (srcs[kx][b, row:row + _RB, _PAD:_PAD + Wo, :]
                     .reshape(M, Cc))
                acc = acc + jnp.dot(a, taps[ky * 3 + kx],
                                    preferred_element_type=f32)
        return acc

    for b in range(B):
        for r in range(NCH):
            y = conv_chunk(b, r, cats, Ctot, wuc, buc_ref)
            o_ref[b, r * _RB:(r + 1) * _RB] = y.reshape(_RB, Wo, C)
            mu = jnp.mean(y, axis=-1, keepdims=True)
            var = jnp.mean((y - mu) ** 2, axis=-1, keepdims=True)
            t = ((y - mu) * jax.lax.rsqrt(var + eps) * g_ref[...]
                 + bln_ref[...])
            tv = t.reshape(_RB, Wo, C).astype(bf16)
            for kx in range(3):
                p = _PAD + 1 - kx
                tps[kx][b, 1 + r * _RB:1 + (r + 1) * _RB, p:p + Wo, :] = tv

    for b in range(B):
        for r in range(NCH):
            h = conv_chunk(b, r, tps, C, w1, b1_ref)
            h = jnp.where(h >= 0, h, h * slope)
            hv = h.reshape(_RB, Wo, C).astype(bf16)
            rows = slice(1 + r * _RB, 1 + (r + 1) * _RB)
            for kx in range(3):
                p = _PAD + 1 - kx
                cats[kx][b, rows, p:p + Wo, 0:C] = hv

    def conv2_chunk(b, r, b_ref):
        acc = jnp.broadcast_to(b_ref[...], (M, C)).astype(f32)
        for ky in range(3):
            row = r * _RB + ky
            for kx in range(3):
                a = (cats[kx][b, row:row + _RB, _PAD:_PAD + Wo, 0:C]
                     .reshape(M, C))
                acc = acc + jnp.dot(a, w2[ky * 3 + kx],
                                    preferred_element_type=f32)
        return acc

    for b in range(B):
        for r in range(NCH):
            h = conv2_chunk(b, r, b2_ref)
            sl = slice(r * _RB, (r + 1) * _RB)
            o_ref[b, sl] = (o_ref[b, sl]
                            + h.reshape(_RB, Wo, C)).astype(o_ref.dtype)


def kernel(x, bridge, w_up, b_up, w_uc, b_uc, ln_g, ln_b, w1, b1, w2, b2):
    N, H, W, Cin = x.shape
    C = w_up.shape[-1]
    Cb = bridge.shape[-1]
    Ho, Wo = 2 * H, 2 * W
    Ctot = C + Cb

    wup_p = w_up.reshape(4 * Cin, C)
    bup_p = b_up.reshape(1, C)
    wuc_p = w_uc.reshape(9 * Ctot, C)
    buc_p = b_uc.reshape(1, C)
    g_p = ln_g.reshape(1, C)
    bln_p = ln_b.reshape(1, C)
    w1_p = w1.reshape(9 * C, C)
    b1_p = b1.reshape(1, C)
    w2_p = w2.reshape(9 * C, C)
    b2_p = b2.reshape(1, C)

    B = _B if N % _B == 0 else 1
    img = lambda n: (n, 0, 0, 0)
    wgt = lambda n: (0, 0)
    return pl.pallas_call(
        functools.partial(_fused_kernel, slope=0.2, eps=1e-5),
        out_shape=jax.ShapeDtypeStruct((N, Ho, Wo, C), x.dtype),
        grid=(N // B,),
        in_specs=[
            pl.BlockSpec((B, H, W, Cin), img),
            pl.BlockSpec((B, Ho, Wo, Cb), img),
            pl.BlockSpec((4 * Cin, C), wgt),
            pl.BlockSpec((1, C), wgt),
            pl.BlockSpec((9 * Ctot, C), wgt),
            pl.BlockSpec((1, C), wgt),
            pl.BlockSpec((1, C), wgt),
            pl.BlockSpec((1, C), wgt),
            pl.BlockSpec((9 * C, C), wgt),
            pl.BlockSpec((1, C), wgt),
            pl.BlockSpec((9 * C, C), wgt),
            pl.BlockSpec((1, C), wgt),
        ],
        out_specs=pl.BlockSpec((B, Ho, Wo, C), img),
        scratch_shapes=[
            pltpu.VMEM((B, H, 2, W, 2 * C), jnp.bfloat16),
            pltpu.VMEM((B, Ho + 2, _COLS, Ctot), jnp.bfloat16),
            pltpu.VMEM((B, Ho + 2, _COLS, Ctot), jnp.bfloat16),
            pltpu.VMEM((B, Ho + 2, _COLS, Ctot), jnp.bfloat16),
            pltpu.VMEM((B, Ho + 2, _COLS, C), jnp.bfloat16),
            pltpu.VMEM((B, Ho + 2, _COLS, C), jnp.bfloat16),
            pltpu.VMEM((B, Ho + 2, _COLS, C), jnp.bfloat16),
        ],
        compiler_params=pltpu.CompilerParams(
            dimension_semantics=("arbitrary",)),
    )(x, bridge, wup_p, bup_p, wuc_p, buc_p, g_p, bln_p, w1_p, b1_p, w2_p, b2_p)
